# Initial kernel scaffold; baseline (speedup 1.0000x reference)
import jax
import jax.numpy as jnp
from jax.experimental import pallas as pl


def kernel(hidden_states, edge_index, W1, b1, W2, b2, W3, b3, Wc, bc):
    raise NotImplementedError("write your pallas kernel here")


# speedup vs baseline: 6.8893x; 6.8893x over previous
import functools

import jax
import jax.numpy as jnp
from jax import lax
from jax.experimental import pallas as pl
from jax.experimental.pallas import tpu as pltpu
from jax.experimental.pallas import tpu_sc as plsc

NC = 2
NS = 16
NR = 4
BK = 128


def _make_deg_kernel(N, EM):
    rows_per_tile = EM // (NC * NS)
    mesh = plsc.VectorSubcoreMesh(core_axis_name="c", subcore_axis_name="s")

    @functools.partial(
        pl.kernel,
        out_type=jax.ShapeDtypeStruct((NC * NS, N), jnp.float32),
        mesh=mesh,
        compiler_params=pltpu.CompilerParams(needs_layout_passes=False),
        scratch_types=[
            pltpu.VMEM((rows_per_tile, 1, 128), jnp.int32),
            pltpu.VMEM((N,), jnp.float32),
        ],
    )
    def deg_kernel(dstm, out, dst_v, deg_l):
        c = lax.axis_index("c")
        s = lax.axis_index("s")
        tile = c * NS + s

        def zero_body(i, _):
            deg_l[pl.ds(i * 16, 16)] = jnp.zeros((16,), jnp.float32)
            return 0

        lax.fori_loop(0, N // 16, zero_body, 0)

        pltpu.sync_copy(dstm.at[pl.ds(tile * rows_per_tile, rows_per_tile)], dst_v)
        ones = jnp.ones((16,), jnp.float32)

        def row_body(j, _):
            def grp_body(g, _):
                idx = dst_v[j, 0, pl.ds(g * 16, 16)]
                plsc.addupdate_scatter(deg_l, [idx], ones)
                return 0

            lax.fori_loop(0, 8, grp_body, 0)
            return 0

        lax.fori_loop(0, rows_per_tile, row_body, 0)
        pltpu.sync_copy(deg_l, out.at[tile])

    return deg_kernel


def _make_agg_kernel(N, E, H):
    per_tile = E // NS
    NQ = 4
    qe = per_tile // NQ
    nblk = qe // BK
    RS = N // NR
    wr = RS // NS
    ZR = 16
    mesh = plsc.VectorSubcoreMesh(core_axis_name="c", subcore_axis_name="s")

    @functools.partial(
        pl.kernel,
        out_type=jax.ShapeDtypeStruct((N, H), jnp.float32),
        mesh=mesh,
        scratch_types=[
            pltpu.VMEM((qe,), jnp.int32),
            pltpu.VMEM((qe // 128, 1, 128), jnp.int32),
            pltpu.VMEM((BK,), jnp.int32),
            pltpu.VMEM((BK, H), jnp.float32),
            pltpu.VMEM((ZR, H), jnp.float32),
            pltpu.VMEM_SHARED((RS + 8, H), jnp.float32),
            pltpu.SemaphoreType.DMA,
        ],
    )
    def agg_kernel(u, src, dstm, out, sidx_v, dst_v, didx_v, rows_v, zer_v,
                   acc_s, sem):
        c = lax.axis_index("c")
        s = lax.axis_index("s")

        def zzero(i, _):
            def zrow(g, _):
                zer_v[i, pl.ds(g * 16, 16)] = jnp.zeros((16,), jnp.float32)
                return 0

            lax.fori_loop(0, H // 16, zrow, 0)
            return 0

        lax.fori_loop(0, ZR, zzero, 0)

        for p in range(NR // NC):
            r = p * NC + c
            base = r * RS

            def acc_zero(zi, _):
                pltpu.sync_copy(zer_v, acc_s.at[pl.ds(s * wr + zi * ZR, ZR)])
                return 0

            lax.fori_loop(0, wr // ZR, acc_zero, 0)

            @pl.when(s == 0)
            def _():
                pltpu.sync_copy(zer_v.at[pl.ds(0, 8)], acc_s.at[pl.ds(RS, 8)])

            plsc.subcore_barrier()

            def chunk(q, _):
                ebase = s * per_tile + q * qe
                pltpu.sync_copy(src.at[pl.ds(ebase, qe)], sidx_v)
                pltpu.sync_copy(dstm.at[pl.ds(ebase // 128, qe // 128)], dst_v)

                def blk(j, _):
                    def grp(g, _):
                        d = dst_v[j, 0, pl.ds(g * 16, 16)]
                        dl = d - base
                        ok = (dl >= 0) & (dl < RS)
                        didx_v[pl.ds(g * 16, 16)] = jnp.where(ok, dl, RS)
                        return 0

                    lax.fori_loop(0, BK // 16, grp, 0)
                    pltpu.async_copy(u.at[sidx_v.at[pl.ds(j * BK, BK)]],
                                     rows_v, sem).wait()
                    pltpu.sync_copy(rows_v, acc_s.at[didx_v], add=True)
                    return 0

                lax.fori_loop(0, nblk, blk, 0)
                return 0

            lax.fori_loop(0, NQ, chunk, 0)
            plsc.subcore_barrier()
            pltpu.sync_copy(acc_s.at[pl.ds(s * wr, wr)],
                            out.at[pl.ds(base + s * wr, wr)])
            if p + 1 < NR // NC:
                plsc.subcore_barrier()

    return agg_kernel


def _dinv_body(degp_ref, dinv_ref, dm32_ref):
    d = jnp.sum(degp_ref[...], axis=0, keepdims=True) + 1.0
    di = lax.rsqrt(d)
    dinv_ref[...] = di
    dm32_ref[...] = di * di * di


def _mm1_body(x_ref, w_ref, dinv_ref, u_ref):
    u_ref[...] = dinv_ref[...] * jnp.dot(x_ref[...], w_ref[...],
                                         preferred_element_type=jnp.float32)


def _mid_body(agg_ref, u_ref, dm32_ref, b_ref, w_ref, dinv_ref, o_ref):
    x = jnp.tanh(dm32_ref[...] * (agg_ref[...] + u_ref[...]) + b_ref[...])
    o_ref[...] = dinv_ref[...] * jnp.dot(x, w_ref[...],
                                         preferred_element_type=jnp.float32)


def _last_body(agg_ref, u_ref, dm32_ref, b_ref, wc_ref, bc_ref, o_ref):
    x = jnp.tanh(dm32_ref[...] * (agg_ref[...] + u_ref[...]) + b_ref[...])
    o_ref[...] = jnp.dot(x, wc_ref[...],
                         preferred_element_type=jnp.float32) + bc_ref[...]


def kernel(hidden_states, edge_index, W1, b1, W2, b2, W3, b3, Wc, bc):
    B, L, D = hidden_states.shape
    N = B * L
    H = W1.shape[0]
    E = edge_index.shape[1]
    O = Wc.shape[1]
    assert H == D and E % (128 * NC * NS) == 0 and N % (NR * NS * 128) == 0

    x = hidden_states.reshape(N, D)
    src = edge_index[0]
    dstm = edge_index[1].reshape(E // 128, 1, 128)

    deg_kernel = _make_deg_kernel(N, E // 128)
    agg_kernel = _make_agg_kernel(N, E, H)

    degp = deg_kernel(dstm)

    bnd = 8192
    dinv_r, dm32_r = pl.pallas_call(
        _dinv_body,
        grid=(N // bnd,),
        in_specs=[pl.BlockSpec((NC * NS, bnd), lambda i: (0, i))],
        out_specs=[pl.BlockSpec((1, bnd), lambda i: (0, i))] * 2,
        out_shape=[jax.ShapeDtypeStruct((1, N), jnp.float32)] * 2,
    )(degp)
    dinv = dinv_r.reshape(N, 1)
    dm32 = dm32_r.reshape(N, 1)

    bn = 1024
    grid = (N // bn,)
    w_spec = pl.BlockSpec((H, H), lambda i: (0, 0))
    row_spec = pl.BlockSpec((1, H), lambda i: (0, 0))
    col_spec = pl.BlockSpec((bn, 1), lambda i: (i, 0))
    x_spec = pl.BlockSpec((bn, H), lambda i: (i, 0))
    u_shape = jax.ShapeDtypeStruct((N, H), jnp.float32)

    u = pl.pallas_call(
        _mm1_body,
        grid=grid,
        in_specs=[x_spec, w_spec, col_spec],
        out_specs=x_spec,
        out_shape=u_shape,
    )(x, W1, dinv)

    mid_call = pl.pallas_call(
        _mid_body,
        grid=grid,
        in_specs=[x_spec, x_spec, col_spec, row_spec, w_spec, col_spec],
        out_specs=x_spec,
        out_shape=u_shape,
    )

    for Wn, bp in ((W2, b1), (W3, b2)):
        agg = agg_kernel(u, src, dstm)
        u = mid_call(agg, u, dm32, bp.reshape(1, H), Wn, dinv)

    agg = agg_kernel(u, src, dstm)
    wc_pad = jnp.pad(Wc, ((0, 0), (0, H - O)))
    bc_pad = jnp.pad(bc, (0, H - O)).reshape(1, H)
    logits_pad = pl.pallas_call(
        _last_body,
        grid=grid,
        in_specs=[x_spec, x_spec, col_spec, row_spec, w_spec, row_spec],
        out_specs=x_spec,
        out_shape=jax.ShapeDtypeStruct((N, H), jnp.float32),
    )(agg, u, dm32, b3.reshape(1, H), wc_pad, bc_pad)

    return logits_pad[:, :O].reshape(B, L, O)


# speedup vs baseline: 7.3769x; 1.0708x over previous
import functools

import jax
import jax.numpy as jnp
from jax import lax
from jax.experimental import pallas as pl
from jax.experimental.pallas import tpu as pltpu
from jax.experimental.pallas import tpu_sc as plsc

NC = 2
NS = 16
NR = 4
BK = 128


def _make_deg_kernel(N, EM):
    rows_per_tile = EM // (NC * NS)
    mesh = plsc.VectorSubcoreMesh(core_axis_name="c", subcore_axis_name="s")

    @functools.partial(
        pl.kernel,
        out_type=jax.ShapeDtypeStruct((NC * NS, N), jnp.float32),
        mesh=mesh,
        compiler_params=pltpu.CompilerParams(needs_layout_passes=False),
        scratch_types=[
            pltpu.VMEM((rows_per_tile, 1, 128), jnp.int32),
            pltpu.VMEM((N,), jnp.float32),
        ],
    )
    def deg_kernel(dstm, out, dst_v, deg_l):
        c = lax.axis_index("c")
        s = lax.axis_index("s")
        tile = c * NS + s

        def zero_body(i, _):
            deg_l[pl.ds(i * 16, 16)] = jnp.zeros((16,), jnp.float32)
            return 0

        lax.fori_loop(0, N // 16, zero_body, 0)

        pltpu.sync_copy(dstm.at[pl.ds(tile * rows_per_tile, rows_per_tile)], dst_v)
        ones = jnp.ones((16,), jnp.float32)

        def row_body(j, _):
            def grp_body(g, _):
                idx = dst_v[j, 0, pl.ds(g * 16, 16)]
                plsc.addupdate_scatter(deg_l, [idx], ones)
                return 0

            lax.fori_loop(0, 8, grp_body, 0)
            return 0

        lax.fori_loop(0, rows_per_tile, row_body, 0)
        pltpu.sync_copy(deg_l, out.at[tile])

    return deg_kernel


def _make_agg_kernel(N, E, H):
    per_tile = E // NS
    NQ = 8
    qe = per_tile // NQ
    nblk = qe // BK
    RS = N // NR
    wr = RS // NS
    ZR = 16
    mesh = plsc.VectorSubcoreMesh(core_axis_name="c", subcore_axis_name="s")

    @functools.partial(
        pl.kernel,
        out_type=jax.ShapeDtypeStruct((N, H), jnp.float32),
        mesh=mesh,
        scratch_types=[
            pltpu.VMEM((qe,), jnp.int32),
            pltpu.VMEM((qe // 128, 1, 128), jnp.int32),
            pltpu.VMEM((BK,), jnp.int32),
            pltpu.VMEM((2, BK, H), jnp.float32),
            pltpu.VMEM((ZR, H), jnp.float32),
            pltpu.VMEM_SHARED((RS + 8, H), jnp.float32),
            pltpu.SemaphoreType.DMA,
            pltpu.SemaphoreType.DMA,
        ],
    )
    def agg_kernel(u, src, dstm, out, sidx_v, dst_v, didx_v, rows_v, zer_v,
                   acc_s, sem0, sem1):
        c = lax.axis_index("c")
        s = lax.axis_index("s")
        sems = (sem0, sem1)

        def zzero(i, _):
            def zrow(g, _):
                zer_v[i, pl.ds(g * 16, 16)] = jnp.zeros((16,), jnp.float32)
                return 0

            lax.fori_loop(0, H // 16, zrow, 0)
            return 0

        lax.fori_loop(0, ZR, zzero, 0)

        def gather(j, b):
            jc = jnp.minimum(j, nblk - 1)
            pltpu.async_copy(u.at[sidx_v.at[pl.ds(jc * BK, BK)]],
                             rows_v.at[b], sems[b])

        def gwait(b):
            pltpu.make_async_copy(u.at[sidx_v.at[pl.ds(0, BK)]],
                                  rows_v.at[b], sems[b]).wait()

        def scatter_blk(j, b, base):
            def grp(g, _):
                d = dst_v[j, 0, pl.ds(g * 16, 16)]
                dl = d - base
                ok = (dl >= 0) & (dl < RS)
                didx_v[pl.ds(g * 16, 16)] = jnp.where(ok, dl, RS)
                return 0

            lax.fori_loop(0, BK // 16, grp, 0)
            pltpu.sync_copy(rows_v.at[b], acc_s.at[didx_v], add=True)

        for p in range(NR // NC):
            r = p * NC + c
            base = r * RS

            def acc_zero(zi, _):
                pltpu.sync_copy(zer_v, acc_s.at[pl.ds(s * wr + zi * ZR, ZR)])
                return 0

            lax.fori_loop(0, wr // ZR, acc_zero, 0)

            @pl.when(s == 0)
            def _():
                pltpu.sync_copy(zer_v.at[pl.ds(0, 8)], acc_s.at[pl.ds(RS, 8)])

            plsc.subcore_barrier()

            def chunk(q, _):
                ebase = s * per_tile + q * qe
                pltpu.sync_copy(src.at[pl.ds(ebase, qe)], sidx_v)
                pltpu.sync_copy(dstm.at[pl.ds(ebase // 128, qe // 128)], dst_v)

                gather(0, 0)

                def blk2(j2, _):
                    j = j2 * 2
                    gather(j + 1, 1)
                    gwait(0)
                    scatter_blk(j, 0, base)
                    gather(j + 2, 0)
                    gwait(1)
                    scatter_blk(j + 1, 1, base)
                    return 0

                lax.fori_loop(0, nblk // 2, blk2, 0)
                gwait(0)
                return 0

            lax.fori_loop(0, NQ, chunk, 0)
            plsc.subcore_barrier()
            pltpu.sync_copy(acc_s.at[pl.ds(s * wr, wr)],
                            out.at[pl.ds(base + s * wr, wr)])
            if p + 1 < NR // NC:
                plsc.subcore_barrier()

    return agg_kernel


def _dinv_body(degp_ref, dinv_ref, dm32_ref):
    d = jnp.sum(degp_ref[...], axis=0, keepdims=True) + 1.0
    di = lax.rsqrt(d)
    dinv_ref[...] = di
    dm32_ref[...] = di * di * di


def _mm1_body(x_ref, w_ref, dinv_ref, u_ref):
    u_ref[...] = dinv_ref[...] * jnp.dot(x_ref[...], w_ref[...],
                                         preferred_element_type=jnp.float32)


def _mid_body(agg_ref, u_ref, dm32_ref, b_ref, w_ref, dinv_ref, o_ref):
    x = jnp.tanh(dm32_ref[...] * (agg_ref[...] + u_ref[...]) + b_ref[...])
    o_ref[...] = dinv_ref[...] * jnp.dot(x, w_ref[...],
                                         preferred_element_type=jnp.float32)


def _last_body(agg_ref, u_ref, dm32_ref, b_ref, wc_ref, bc_ref, o_ref):
    x = jnp.tanh(dm32_ref[...] * (agg_ref[...] + u_ref[...]) + b_ref[...])
    o_ref[...] = jnp.dot(x, wc_ref[...],
                         preferred_element_type=jnp.float32) + bc_ref[...]


def kernel(hidden_states, edge_index, W1, b1, W2, b2, W3, b3, Wc, bc):
    B, L, D = hidden_states.shape
    N = B * L
    H = W1.shape[0]
    E = edge_index.shape[1]
    O = Wc.shape[1]
    assert H == D and E % (128 * NC * NS) == 0 and N % (NR * NS * 128) == 0

    x = hidden_states.reshape(N, D)
    src = edge_index[0]
    dstm = edge_index[1].reshape(E // 128, 1, 128)

    deg_kernel = _make_deg_kernel(N, E // 128)
    agg_kernel = _make_agg_kernel(N, E, H)

    degp = deg_kernel(dstm)

    bnd = 8192
    dinv_r, dm32_r = pl.pallas_call(
        _dinv_body,
        grid=(N // bnd,),
        in_specs=[pl.BlockSpec((NC * NS, bnd), lambda i: (0, i))],
        out_specs=[pl.BlockSpec((1, bnd), lambda i: (0, i))] * 2,
        out_shape=[jax.ShapeDtypeStruct((1, N), jnp.float32)] * 2,
    )(degp)
    dinv = dinv_r.reshape(N, 1)
    dm32 = dm32_r.reshape(N, 1)

    bn = 1024
    grid = (N // bn,)
    w_spec = pl.BlockSpec((H, H), lambda i: (0, 0))
    row_spec = pl.BlockSpec((1, H), lambda i: (0, 0))
    col_spec = pl.BlockSpec((bn, 1), lambda i: (i, 0))
    x_spec = pl.BlockSpec((bn, H), lambda i: (i, 0))
    u_shape = jax.ShapeDtypeStruct((N, H), jnp.float32)

    u = pl.pallas_call(
        _mm1_body,
        grid=grid,
        in_specs=[x_spec, w_spec, col_spec],
        out_specs=x_spec,
        out_shape=u_shape,
    )(x, W1, dinv)

    mid_call = pl.pallas_call(
        _mid_body,
        grid=grid,
        in_specs=[x_spec, x_spec, col_spec, row_spec, w_spec, col_spec],
        out_specs=x_spec,
        out_shape=u_shape,
    )

    for Wn, bp in ((W2, b1), (W3, b2)):
        agg = agg_kernel(u, src, dstm)
        u = mid_call(agg, u, dm32, bp.reshape(1, H), Wn, dinv)

    agg = agg_kernel(u, src, dstm)
    wc_pad = jnp.pad(Wc, ((0, 0), (0, H - O)))
    bc_pad = jnp.pad(bc, (0, H - O)).reshape(1, H)
    logits_pad = pl.pallas_call(
        _last_body,
        grid=grid,
        in_specs=[x_spec, x_spec, col_spec, row_spec, w_spec, row_spec],
        out_specs=x_spec,
        out_shape=jax.ShapeDtypeStruct((N, H), jnp.float32),
    )(agg, u, dm32, b3.reshape(1, H), wc_pad, bc_pad)

    return logits_pad[:, :O].reshape(B, L, O)
